# baseline (speedup 1.0000x reference)
import functools

import jax
import jax.numpy as jnp
from jax import lax
from jax.experimental import pallas as pl
from jax.experimental.pallas import tpu as pltpu
from jax.experimental.pallas import tpu_sc as plsc

N_ATOMS = 10000
N_PAD = 12288
NC, NS = 2, 16
CHUNK = N_PAD // (NC * NS)
NOSEL = 2048

N_NEURAL = 2048
N_B = 4096
N_CLAUSE = 1024
NEU_T = N_NEURAL // NS
B_T = N_B // NS
CL_T = N_CLAUSE // NS

BATCH = 4096
D_Z = 128
BR = 256
BC = 2048


def _sc_body(nidx_hbm, bidx_hbm, cidx_hbm, cw_hbm, neg1_hbm, zero_hbm,
             sel_hbm, cvec_hbm,
             nidx_v, bidx_v, cidx_v, cw_v,
             lastk_v, addv_v, tmp_v, tmpf_v, acc_v, a_v, c_v, g_v, sem,
             lk_slots, av_slots):
    cid = lax.axis_index("c")
    sid = lax.axis_index("s")
    base = (cid * NS + sid) * CHUNK
    lane = lax.iota(jnp.int32, 16)

    pltpu.sync_copy(nidx_hbm.at[pl.ds(sid * NEU_T, NEU_T)], nidx_v)
    pltpu.sync_copy(bidx_hbm.at[pl.ds(sid * B_T, B_T)], bidx_v)
    pltpu.sync_copy(cidx_hbm.at[pl.ds(sid * CL_T, CL_T)], cidx_v)
    pltpu.sync_copy(cw_hbm.at[pl.ds(sid * CL_T, CL_T)], cw_v)
    pltpu.sync_copy(neg1_hbm, lastk_v)
    pltpu.sync_copy(zero_hbm, addv_v)

    ones16 = jnp.ones((16,), jnp.float32)
    for v in range(B_T // 16):
        plsc.addupdate_scatter(addv_v, [bidx_v[pl.ds(v * 16, 16)]], ones16)
    for v in range(CL_T // 16):
        plsc.addupdate_scatter(addv_v, [cidx_v[pl.ds(v * 16, 16)]],
                               cw_v[pl.ds(v * 16, 16)])

    for v in range(NEU_T // 16):
        idx = nidx_v[pl.ds(v * 16, 16)]
        kvec = lane + (sid * NEU_T + v * 16)
        m = lane < 16
        for s in range(1, 16):
            j = v * 16 + jnp.minimum(lane + s, 15)
            g = plsc.load_gather(nidx_v, [j])
            m = m & ((g != idx) | (lane + s > 15))
        plsc.store_scatter(lastk_v, [idx], kvec, mask=m)

    pltpu.sync_copy(lastk_v, lk_slots.at[sid])
    pltpu.sync_copy(addv_v, av_slots.at[sid])
    plsc.subcore_barrier()
    descs = []
    for s in range(NS):
        descs.append(pltpu.async_copy(
            lk_slots.at[s, pl.ds(base, CHUNK)], tmp_v.at[s], sem))
        descs.append(pltpu.async_copy(
            av_slots.at[s, pl.ds(base, CHUNK)], tmpf_v.at[s], sem))
    for d in descs:
        d.wait()
    for t in range(CHUNK // 16):
        r = tmp_v[0, pl.ds(t * 16, 16)]
        af = tmpf_v[0, pl.ds(t * 16, 16)]
        for s in range(1, NS):
            r = jnp.maximum(r, tmp_v[s, pl.ds(t * 16, 16)])
            af = af + tmpf_v[s, pl.ds(t * 16, 16)]
        acc_v[pl.ds(t * 16, 16)] = r
        a_v[pl.ds(t * 16, 16)] = af

    for t in range(CHUNK // 16):
        sel = acc_v[pl.ds(t * 16, 16)]
        a = a_v[pl.ds(t * 16, 16)]
        atom = base + t * 16 + lane
        isn = sel >= 0
        c = jnp.where(isn, a + 0.5, a)
        g = jnp.where(isn, sel, NOSEL)
        is1 = atom == 1
        c = jnp.where(is1, 1.0, c)
        g = jnp.where(is1, NOSEL, g)
        c_v[pl.ds(t * 16, 16)] = c
        g_v[pl.ds(t * 16, 16)] = g

    pltpu.sync_copy(c_v, cvec_hbm.at[pl.ds(base, CHUNK)])
    pltpu.sync_copy(g_v, sel_hbm.at[pl.ds(base, CHUNK)])


@functools.partial(
    pl.kernel,
    out_type=(jax.ShapeDtypeStruct((N_PAD,), jnp.int32),
              jax.ShapeDtypeStruct((N_PAD,), jnp.float32)),
    mesh=plsc.VectorSubcoreMesh(core_axis_name="c", subcore_axis_name="s",
                                num_cores=NC, num_subcores=NS),
    compiler_params=pltpu.CompilerParams(needs_layout_passes=False),
    scratch_types=(
        pltpu.VMEM((NEU_T,), jnp.int32),
        pltpu.VMEM((B_T,), jnp.int32),
        pltpu.VMEM((CL_T,), jnp.int32),
        pltpu.VMEM((CL_T,), jnp.float32),
        pltpu.VMEM((N_PAD,), jnp.int32),
        pltpu.VMEM((N_PAD,), jnp.float32),
        pltpu.VMEM((NS, CHUNK), jnp.int32),
        pltpu.VMEM((NS, CHUNK), jnp.float32),
        pltpu.VMEM((CHUNK,), jnp.int32),
        pltpu.VMEM((CHUNK,), jnp.float32),
        pltpu.VMEM((CHUNK,), jnp.float32),
        pltpu.VMEM((CHUNK,), jnp.int32),
        pltpu.SemaphoreType.DMA,
        pltpu.VMEM_SHARED((NS, N_PAD), jnp.int32),
        pltpu.VMEM_SHARED((NS, N_PAD), jnp.float32),
    ),
)
def _sc_build(nidx_hbm, bidx_hbm, cidx_hbm, cw_hbm, neg1_hbm, zero_hbm,
              sel_hbm, cvec_hbm, *scratch):
    _sc_body(nidx_hbm, bidx_hbm, cidx_hbm, cw_hbm, neg1_hbm, zero_hbm,
             sel_hbm, cvec_hbm, *scratch)


def _tc_body(z_ref, w_ref, sel_ref, c_ref, o_ref, wsel_s):
    @pl.when(pl.program_id(1) == 0)
    def _():
        sel = sel_ref[0, :]
        kio = lax.broadcasted_iota(jnp.int32, (BC, N_NEURAL), 1)
        onehot = jnp.where(kio == sel[:, None], 0.5, 0.0).astype(jnp.bfloat16)
        wb = w_ref[...].astype(jnp.bfloat16)
        wsel_s[...] = lax.dot_general(
            onehot, wb, (((1,), (1,)), ((), ())),
            preferred_element_type=jnp.float32).astype(jnp.bfloat16)

    acc = lax.dot_general(z_ref[...].astype(jnp.bfloat16), wsel_s[...],
                          (((1,), (1,)), ((), ())),
                          preferred_element_type=jnp.float32)
    o_ref[...] = jnp.tanh(acc) * 0.5 + c_ref[...]


_tc_apply = pl.pallas_call(
    _tc_body,
    grid=(10240 // BC, BATCH // BR),
    in_specs=[
        pl.BlockSpec((BR, D_Z), lambda j, i: (i, 0)),
        pl.BlockSpec((D_Z, N_NEURAL), lambda j, i: (0, 0)),
        pl.BlockSpec((1, BC), lambda j, i: (0, j)),
        pl.BlockSpec((1, BC), lambda j, i: (0, j)),
    ],
    out_specs=pl.BlockSpec((BR, BC), lambda j, i: (i, j)),
    out_shape=jax.ShapeDtypeStruct((BATCH, N_ATOMS), jnp.float32),
    scratch_shapes=[pltpu.VMEM((BC, D_Z), jnp.bfloat16)],
)


def kernel(Z, W_vm, clause_weights, neural_idx, b_idx, clause_idx):
    neg1 = jnp.full((N_PAD,), -1, jnp.int32)
    zero = jnp.zeros((N_PAD,), jnp.float32)
    sel, cvec = _sc_build(neural_idx, b_idx, clause_idx, clause_weights,
                          neg1, zero)
    return _tc_apply(Z, W_vm, sel.reshape(1, N_PAD), cvec.reshape(1, N_PAD))


# speedup vs baseline: 1.0848x; 1.0848x over previous
import functools

import jax
import jax.numpy as jnp
from jax import lax
from jax.experimental import pallas as pl
from jax.experimental.pallas import tpu as pltpu
from jax.experimental.pallas import tpu_sc as plsc

N_ATOMS = 10000
N_PAD = 12288
NC, NS = 2, 16
CHUNK = N_PAD // (NC * NS)
NOSEL = 2048

N_NEURAL = 2048
N_B = 4096
N_CLAUSE = 1024
NEU_T = N_NEURAL // NS
B_T = N_B // NS
CL_T = N_CLAUSE // NS

BATCH = 4096
D_Z = 128
BR = 512
BC = 2048


def _sc_body(nidx_hbm, bidx_hbm, cidx_hbm, cw_hbm, neg1_hbm, zero_hbm,
             sel_hbm, cvec_hbm,
             nidx_v, bidx_v, cidx_v, cw_v,
             lastk_v, addv_v, tmp_v, tmpf_v, acc_v, a_v, c_v, g_v, sem,
             lk_slots, av_slots):
    cid = lax.axis_index("c")
    sid = lax.axis_index("s")
    base = (cid * NS + sid) * CHUNK
    lane = lax.iota(jnp.int32, 16)

    pltpu.sync_copy(nidx_hbm.at[pl.ds(sid * NEU_T, NEU_T)], nidx_v)
    pltpu.sync_copy(bidx_hbm.at[pl.ds(sid * B_T, B_T)], bidx_v)
    pltpu.sync_copy(cidx_hbm.at[pl.ds(sid * CL_T, CL_T)], cidx_v)
    pltpu.sync_copy(cw_hbm.at[pl.ds(sid * CL_T, CL_T)], cw_v)
    pltpu.sync_copy(neg1_hbm, lastk_v)
    pltpu.sync_copy(zero_hbm, addv_v)

    ones16 = jnp.ones((16,), jnp.float32)
    for v in range(B_T // 16):
        plsc.addupdate_scatter(addv_v, [bidx_v[pl.ds(v * 16, 16)]], ones16)
    for v in range(CL_T // 16):
        plsc.addupdate_scatter(addv_v, [cidx_v[pl.ds(v * 16, 16)]],
                               cw_v[pl.ds(v * 16, 16)])

    for v in range(NEU_T // 16):
        idx = nidx_v[pl.ds(v * 16, 16)]
        kvec = lane + (sid * NEU_T + v * 16)
        m = lane < 16
        for s in range(1, 16):
            j = v * 16 + jnp.minimum(lane + s, 15)
            g = plsc.load_gather(nidx_v, [j])
            m = m & ((g != idx) | (lane + s > 15))
        plsc.store_scatter(lastk_v, [idx], kvec, mask=m)

    pltpu.sync_copy(lastk_v, lk_slots.at[sid])
    pltpu.sync_copy(addv_v, av_slots.at[sid])
    plsc.subcore_barrier()
    descs = []
    for s in range(NS):
        descs.append(pltpu.async_copy(
            lk_slots.at[s, pl.ds(base, CHUNK)], tmp_v.at[s], sem))
        descs.append(pltpu.async_copy(
            av_slots.at[s, pl.ds(base, CHUNK)], tmpf_v.at[s], sem))
    for d in descs:
        d.wait()
    for t in range(CHUNK // 16):
        r = tmp_v[0, pl.ds(t * 16, 16)]
        af = tmpf_v[0, pl.ds(t * 16, 16)]
        for s in range(1, NS):
            r = jnp.maximum(r, tmp_v[s, pl.ds(t * 16, 16)])
            af = af + tmpf_v[s, pl.ds(t * 16, 16)]
        acc_v[pl.ds(t * 16, 16)] = r
        a_v[pl.ds(t * 16, 16)] = af

    for t in range(CHUNK // 16):
        sel = acc_v[pl.ds(t * 16, 16)]
        a = a_v[pl.ds(t * 16, 16)]
        atom = base + t * 16 + lane
        isn = sel >= 0
        c = jnp.where(isn, a + 0.5, a)
        g = jnp.where(isn, sel, NOSEL)
        is1 = atom == 1
        c = jnp.where(is1, 1.0, c)
        g = jnp.where(is1, NOSEL, g)
        c_v[pl.ds(t * 16, 16)] = c
        g_v[pl.ds(t * 16, 16)] = g

    pltpu.sync_copy(c_v, cvec_hbm.at[pl.ds(base, CHUNK)])
    pltpu.sync_copy(g_v, sel_hbm.at[pl.ds(base, CHUNK)])


@functools.partial(
    pl.kernel,
    out_type=(jax.ShapeDtypeStruct((N_PAD,), jnp.int32),
              jax.ShapeDtypeStruct((N_PAD,), jnp.float32)),
    mesh=plsc.VectorSubcoreMesh(core_axis_name="c", subcore_axis_name="s",
                                num_cores=NC, num_subcores=NS),
    compiler_params=pltpu.CompilerParams(needs_layout_passes=False),
    scratch_types=(
        pltpu.VMEM((NEU_T,), jnp.int32),
        pltpu.VMEM((B_T,), jnp.int32),
        pltpu.VMEM((CL_T,), jnp.int32),
        pltpu.VMEM((CL_T,), jnp.float32),
        pltpu.VMEM((N_PAD,), jnp.int32),
        pltpu.VMEM((N_PAD,), jnp.float32),
        pltpu.VMEM((NS, CHUNK), jnp.int32),
        pltpu.VMEM((NS, CHUNK), jnp.float32),
        pltpu.VMEM((CHUNK,), jnp.int32),
        pltpu.VMEM((CHUNK,), jnp.float32),
        pltpu.VMEM((CHUNK,), jnp.float32),
        pltpu.VMEM((CHUNK,), jnp.int32),
        pltpu.SemaphoreType.DMA,
        pltpu.VMEM_SHARED((NS, N_PAD), jnp.int32),
        pltpu.VMEM_SHARED((NS, N_PAD), jnp.float32),
    ),
)
def _sc_build(nidx_hbm, bidx_hbm, cidx_hbm, cw_hbm, neg1_hbm, zero_hbm,
              sel_hbm, cvec_hbm, *scratch):
    _sc_body(nidx_hbm, bidx_hbm, cidx_hbm, cw_hbm, neg1_hbm, zero_hbm,
             sel_hbm, cvec_hbm, *scratch)


def _tc_body(z_ref, w_ref, sel_ref, c_ref, o_ref, wsel_s):
    @pl.when(pl.program_id(1) == 0)
    def _():
        sel = sel_ref[0, :]
        kio = lax.broadcasted_iota(jnp.int32, (BC, N_NEURAL), 1)
        onehot = jnp.where(kio == sel[:, None], 0.5, 0.0).astype(jnp.bfloat16)
        wb = w_ref[...].astype(jnp.bfloat16)
        wsel_s[...] = lax.dot_general(
            onehot, wb, (((1,), (1,)), ((), ())),
            preferred_element_type=jnp.float32).astype(jnp.bfloat16)

    acc = lax.dot_general(z_ref[...].astype(jnp.bfloat16), wsel_s[...],
                          (((1,), (1,)), ((), ())),
                          preferred_element_type=jnp.float32)
    o_ref[...] = jnp.tanh(acc) * 0.5 + c_ref[...]


_tc_apply = pl.pallas_call(
    _tc_body,
    grid=(10240 // BC, BATCH // BR),
    in_specs=[
        pl.BlockSpec((BR, D_Z), lambda j, i: (i, 0)),
        pl.BlockSpec((D_Z, N_NEURAL), lambda j, i: (0, 0)),
        pl.BlockSpec((1, BC), lambda j, i: (0, j)),
        pl.BlockSpec((1, BC), lambda j, i: (0, j)),
    ],
    out_specs=pl.BlockSpec((BR, BC), lambda j, i: (i, j)),
    out_shape=jax.ShapeDtypeStruct((BATCH, N_ATOMS), jnp.float32),
    scratch_shapes=[pltpu.VMEM((BC, D_Z), jnp.bfloat16)],
)


def kernel(Z, W_vm, clause_weights, neural_idx, b_idx, clause_idx):
    neg1 = jnp.full((N_PAD,), -1, jnp.int32)
    zero = jnp.zeros((N_PAD,), jnp.float32)
    sel, cvec = _sc_build(neural_idx, b_idx, clause_idx, clause_weights,
                          neg1, zero)
    return _tc_apply(Z, W_vm, sel.reshape(1, N_PAD), cvec.reshape(1, N_PAD))


# speedup vs baseline: 1.1257x; 1.0377x over previous
import functools

import jax
import jax.numpy as jnp
from jax import lax
from jax.experimental import pallas as pl
from jax.experimental.pallas import tpu as pltpu
from jax.experimental.pallas import tpu_sc as plsc

N_ATOMS = 10000
N_PAD = 12288
NC, NS = 2, 16
CHUNK = N_PAD // (NC * NS)
NOSEL = 2048

N_NEURAL = 2048
N_B = 4096
N_CLAUSE = 1024
NEU_T = N_NEURAL // NS
B_T = N_B // NS
CL_T = N_CLAUSE // NS

BATCH = 4096
D_Z = 128
BR = 1024
BC = 2048


def _sc_body(nidx_hbm, bidx_hbm, cidx_hbm, cw_hbm, neg1_hbm, zero_hbm,
             sel_hbm, cvec_hbm,
             nidx_v, bidx_v, cidx_v, cw_v,
             lastk_v, addv_v, tmp_v, tmpf_v, acc_v, a_v, c_v, g_v, sem,
             lk_slots, av_slots):
    cid = lax.axis_index("c")
    sid = lax.axis_index("s")
    base = (cid * NS + sid) * CHUNK
    lane = lax.iota(jnp.int32, 16)

    pltpu.sync_copy(nidx_hbm.at[pl.ds(sid * NEU_T, NEU_T)], nidx_v)
    pltpu.sync_copy(bidx_hbm.at[pl.ds(sid * B_T, B_T)], bidx_v)
    pltpu.sync_copy(cidx_hbm.at[pl.ds(sid * CL_T, CL_T)], cidx_v)
    pltpu.sync_copy(cw_hbm.at[pl.ds(sid * CL_T, CL_T)], cw_v)
    pltpu.sync_copy(neg1_hbm, lastk_v)
    pltpu.sync_copy(zero_hbm, addv_v)

    ones16 = jnp.ones((16,), jnp.float32)
    for v in range(B_T // 16):
        plsc.addupdate_scatter(addv_v, [bidx_v[pl.ds(v * 16, 16)]], ones16)
    for v in range(CL_T // 16):
        plsc.addupdate_scatter(addv_v, [cidx_v[pl.ds(v * 16, 16)]],
                               cw_v[pl.ds(v * 16, 16)])

    for v in range(NEU_T // 16):
        idx = nidx_v[pl.ds(v * 16, 16)]
        kvec = lane + (sid * NEU_T + v * 16)
        m = lane < 16
        for s in range(1, 16):
            j = v * 16 + jnp.minimum(lane + s, 15)
            g = plsc.load_gather(nidx_v, [j])
            m = m & ((g != idx) | (lane + s > 15))
        plsc.store_scatter(lastk_v, [idx], kvec, mask=m)

    pltpu.sync_copy(lastk_v, lk_slots.at[sid])
    pltpu.sync_copy(addv_v, av_slots.at[sid])
    plsc.subcore_barrier()
    descs = []
    for s in range(NS):
        descs.append(pltpu.async_copy(
            lk_slots.at[s, pl.ds(base, CHUNK)], tmp_v.at[s], sem))
        descs.append(pltpu.async_copy(
            av_slots.at[s, pl.ds(base, CHUNK)], tmpf_v.at[s], sem))
    for d in descs:
        d.wait()
    for t in range(CHUNK // 16):
        r = tmp_v[0, pl.ds(t * 16, 16)]
        af = tmpf_v[0, pl.ds(t * 16, 16)]
        for s in range(1, NS):
            r = jnp.maximum(r, tmp_v[s, pl.ds(t * 16, 16)])
            af = af + tmpf_v[s, pl.ds(t * 16, 16)]
        acc_v[pl.ds(t * 16, 16)] = r
        a_v[pl.ds(t * 16, 16)] = af

    for t in range(CHUNK // 16):
        sel = acc_v[pl.ds(t * 16, 16)]
        a = a_v[pl.ds(t * 16, 16)]
        atom = base + t * 16 + lane
        isn = sel >= 0
        c = jnp.where(isn, a + 0.5, a)
        g = jnp.where(isn, sel, NOSEL)
        is1 = atom == 1
        c = jnp.where(is1, 1.0, c)
        g = jnp.where(is1, NOSEL, g)
        c_v[pl.ds(t * 16, 16)] = c
        g_v[pl.ds(t * 16, 16)] = g

    pltpu.sync_copy(c_v, cvec_hbm.at[pl.ds(base, CHUNK)])
    pltpu.sync_copy(g_v, sel_hbm.at[pl.ds(base, CHUNK)])


@functools.partial(
    pl.kernel,
    out_type=(jax.ShapeDtypeStruct((N_PAD,), jnp.int32),
              jax.ShapeDtypeStruct((N_PAD,), jnp.float32)),
    mesh=plsc.VectorSubcoreMesh(core_axis_name="c", subcore_axis_name="s",
                                num_cores=NC, num_subcores=NS),
    compiler_params=pltpu.CompilerParams(needs_layout_passes=False),
    scratch_types=(
        pltpu.VMEM((NEU_T,), jnp.int32),
        pltpu.VMEM((B_T,), jnp.int32),
        pltpu.VMEM((CL_T,), jnp.int32),
        pltpu.VMEM((CL_T,), jnp.float32),
        pltpu.VMEM((N_PAD,), jnp.int32),
        pltpu.VMEM((N_PAD,), jnp.float32),
        pltpu.VMEM((NS, CHUNK), jnp.int32),
        pltpu.VMEM((NS, CHUNK), jnp.float32),
        pltpu.VMEM((CHUNK,), jnp.int32),
        pltpu.VMEM((CHUNK,), jnp.float32),
        pltpu.VMEM((CHUNK,), jnp.float32),
        pltpu.VMEM((CHUNK,), jnp.int32),
        pltpu.SemaphoreType.DMA,
        pltpu.VMEM_SHARED((NS, N_PAD), jnp.int32),
        pltpu.VMEM_SHARED((NS, N_PAD), jnp.float32),
    ),
)
def _sc_build(nidx_hbm, bidx_hbm, cidx_hbm, cw_hbm, neg1_hbm, zero_hbm,
              sel_hbm, cvec_hbm, *scratch):
    _sc_body(nidx_hbm, bidx_hbm, cidx_hbm, cw_hbm, neg1_hbm, zero_hbm,
             sel_hbm, cvec_hbm, *scratch)


def _tc_body(z_ref, w_ref, sel_ref, c_ref, o_ref, wsel_s):
    @pl.when(pl.program_id(1) == 0)
    def _():
        sel = sel_ref[0, :]
        kio = lax.broadcasted_iota(jnp.int32, (BC, N_NEURAL), 1)
        onehot = jnp.where(kio == sel[:, None], 0.5, 0.0).astype(jnp.bfloat16)
        wb = w_ref[...].astype(jnp.bfloat16)
        wsel_s[...] = lax.dot_general(
            onehot, wb, (((1,), (1,)), ((), ())),
            preferred_element_type=jnp.float32).astype(jnp.bfloat16)

    acc = lax.dot_general(z_ref[...].astype(jnp.bfloat16), wsel_s[...],
                          (((1,), (1,)), ((), ())),
                          preferred_element_type=jnp.float32)
    o_ref[...] = jnp.tanh(acc) * 0.5 + c_ref[...]


_tc_apply = pl.pallas_call(
    _tc_body,
    grid=(10240 // BC, BATCH // BR),
    in_specs=[
        pl.BlockSpec((BR, D_Z), lambda j, i: (i, 0)),
        pl.BlockSpec((D_Z, N_NEURAL), lambda j, i: (0, 0)),
        pl.BlockSpec((1, BC), lambda j, i: (0, j)),
        pl.BlockSpec((1, BC), lambda j, i: (0, j)),
    ],
    out_specs=pl.BlockSpec((BR, BC), lambda j, i: (i, j)),
    out_shape=jax.ShapeDtypeStruct((BATCH, N_ATOMS), jnp.float32),
    scratch_shapes=[pltpu.VMEM((BC, D_Z), jnp.bfloat16)],
)


def kernel(Z, W_vm, clause_weights, neural_idx, b_idx, clause_idx):
    neg1 = jnp.full((N_PAD,), -1, jnp.int32)
    zero = jnp.zeros((N_PAD,), jnp.float32)
    sel, cvec = _sc_build(neural_idx, b_idx, clause_idx, clause_weights,
                          neg1, zero)
    return _tc_apply(Z, W_vm, sel.reshape(1, N_PAD), cvec.reshape(1, N_PAD))


# speedup vs baseline: 1.1413x; 1.0139x over previous
import functools

import jax
import jax.numpy as jnp
from jax import lax
from jax.experimental import pallas as pl
from jax.experimental.pallas import tpu as pltpu
from jax.experimental.pallas import tpu_sc as plsc

N_ATOMS = 10000
N_PAD = 12288
NC, NS = 2, 16
CHUNK = N_PAD // (NC * NS)
NOSEL = 2048

N_NEURAL = 2048
N_B = 4096
N_CLAUSE = 1024
NEU_T = N_NEURAL // NS
B_T = N_B // NS
CL_T = N_CLAUSE // NS

BATCH = 4096
D_Z = 128
BR = 2048
BC = 2048


def _sc_body(nidx_hbm, bidx_hbm, cidx_hbm, cw_hbm, neg1_hbm, zero_hbm,
             sel_hbm, cvec_hbm,
             nidx_v, bidx_v, cidx_v, cw_v,
             lastk_v, addv_v, tmp_v, tmpf_v, acc_v, a_v, c_v, g_v, sem,
             lk_slots, av_slots):
    cid = lax.axis_index("c")
    sid = lax.axis_index("s")
    base = (cid * NS + sid) * CHUNK
    lane = lax.iota(jnp.int32, 16)

    pltpu.sync_copy(nidx_hbm.at[pl.ds(sid * NEU_T, NEU_T)], nidx_v)
    pltpu.sync_copy(bidx_hbm.at[pl.ds(sid * B_T, B_T)], bidx_v)
    pltpu.sync_copy(cidx_hbm.at[pl.ds(sid * CL_T, CL_T)], cidx_v)
    pltpu.sync_copy(cw_hbm.at[pl.ds(sid * CL_T, CL_T)], cw_v)
    pltpu.sync_copy(neg1_hbm, lastk_v)
    pltpu.sync_copy(zero_hbm, addv_v)

    ones16 = jnp.ones((16,), jnp.float32)
    for v in range(B_T // 16):
        plsc.addupdate_scatter(addv_v, [bidx_v[pl.ds(v * 16, 16)]], ones16)
    for v in range(CL_T // 16):
        plsc.addupdate_scatter(addv_v, [cidx_v[pl.ds(v * 16, 16)]],
                               cw_v[pl.ds(v * 16, 16)])

    for v in range(NEU_T // 16):
        idx = nidx_v[pl.ds(v * 16, 16)]
        kvec = lane + (sid * NEU_T + v * 16)
        m = lane < 16
        for s in range(1, 16):
            j = v * 16 + jnp.minimum(lane + s, 15)
            g = plsc.load_gather(nidx_v, [j])
            m = m & ((g != idx) | (lane + s > 15))
        plsc.store_scatter(lastk_v, [idx], kvec, mask=m)

    pltpu.sync_copy(lastk_v, lk_slots.at[sid])
    pltpu.sync_copy(addv_v, av_slots.at[sid])
    plsc.subcore_barrier()
    descs = []
    for s in range(NS):
        descs.append(pltpu.async_copy(
            lk_slots.at[s, pl.ds(base, CHUNK)], tmp_v.at[s], sem))
        descs.append(pltpu.async_copy(
            av_slots.at[s, pl.ds(base, CHUNK)], tmpf_v.at[s], sem))
    for d in descs:
        d.wait()
    for t in range(CHUNK // 16):
        r = tmp_v[0, pl.ds(t * 16, 16)]
        af = tmpf_v[0, pl.ds(t * 16, 16)]
        for s in range(1, NS):
            r = jnp.maximum(r, tmp_v[s, pl.ds(t * 16, 16)])
            af = af + tmpf_v[s, pl.ds(t * 16, 16)]
        acc_v[pl.ds(t * 16, 16)] = r
        a_v[pl.ds(t * 16, 16)] = af

    for t in range(CHUNK // 16):
        sel = acc_v[pl.ds(t * 16, 16)]
        a = a_v[pl.ds(t * 16, 16)]
        atom = base + t * 16 + lane
        isn = sel >= 0
        c = jnp.where(isn, a + 0.5, a)
        g = jnp.where(isn, sel, NOSEL)
        is1 = atom == 1
        c = jnp.where(is1, 1.0, c)
        g = jnp.where(is1, NOSEL, g)
        c_v[pl.ds(t * 16, 16)] = c
        g_v[pl.ds(t * 16, 16)] = g

    pltpu.sync_copy(c_v, cvec_hbm.at[pl.ds(base, CHUNK)])
    pltpu.sync_copy(g_v, sel_hbm.at[pl.ds(base, CHUNK)])


@functools.partial(
    pl.kernel,
    out_type=(jax.ShapeDtypeStruct((N_PAD,), jnp.int32),
              jax.ShapeDtypeStruct((N_PAD,), jnp.float32)),
    mesh=plsc.VectorSubcoreMesh(core_axis_name="c", subcore_axis_name="s",
                                num_cores=NC, num_subcores=NS),
    compiler_params=pltpu.CompilerParams(needs_layout_passes=False),
    scratch_types=(
        pltpu.VMEM((NEU_T,), jnp.int32),
        pltpu.VMEM((B_T,), jnp.int32),
        pltpu.VMEM((CL_T,), jnp.int32),
        pltpu.VMEM((CL_T,), jnp.float32),
        pltpu.VMEM((N_PAD,), jnp.int32),
        pltpu.VMEM((N_PAD,), jnp.float32),
        pltpu.VMEM((NS, CHUNK), jnp.int32),
        pltpu.VMEM((NS, CHUNK), jnp.float32),
        pltpu.VMEM((CHUNK,), jnp.int32),
        pltpu.VMEM((CHUNK,), jnp.float32),
        pltpu.VMEM((CHUNK,), jnp.float32),
        pltpu.VMEM((CHUNK,), jnp.int32),
        pltpu.SemaphoreType.DMA,
        pltpu.VMEM_SHARED((NS, N_PAD), jnp.int32),
        pltpu.VMEM_SHARED((NS, N_PAD), jnp.float32),
    ),
)
def _sc_build(nidx_hbm, bidx_hbm, cidx_hbm, cw_hbm, neg1_hbm, zero_hbm,
              sel_hbm, cvec_hbm, *scratch):
    _sc_body(nidx_hbm, bidx_hbm, cidx_hbm, cw_hbm, neg1_hbm, zero_hbm,
             sel_hbm, cvec_hbm, *scratch)


def _tc_body(z_ref, w_ref, sel_ref, c_ref, o_ref, wsel_s):
    @pl.when(pl.program_id(1) == 0)
    def _():
        sel = sel_ref[0, :]
        kio = lax.broadcasted_iota(jnp.int32, (BC, N_NEURAL), 1)
        onehot = jnp.where(kio == sel[:, None], 0.5, 0.0).astype(jnp.bfloat16)
        wb = w_ref[...].astype(jnp.bfloat16)
        wsel_s[...] = lax.dot_general(
            onehot, wb, (((1,), (1,)), ((), ())),
            preferred_element_type=jnp.float32).astype(jnp.bfloat16)

    acc = lax.dot_general(z_ref[...].astype(jnp.bfloat16), wsel_s[...],
                          (((1,), (1,)), ((), ())),
                          preferred_element_type=jnp.float32)
    o_ref[...] = jnp.tanh(acc) * 0.5 + c_ref[...]


_tc_apply = pl.pallas_call(
    _tc_body,
    grid=(10240 // BC, BATCH // BR),
    in_specs=[
        pl.BlockSpec((BR, D_Z), lambda j, i: (i, 0)),
        pl.BlockSpec((D_Z, N_NEURAL), lambda j, i: (0, 0)),
        pl.BlockSpec((1, BC), lambda j, i: (0, j)),
        pl.BlockSpec((1, BC), lambda j, i: (0, j)),
    ],
    out_specs=pl.BlockSpec((BR, BC), lambda j, i: (i, j)),
    out_shape=jax.ShapeDtypeStruct((BATCH, N_ATOMS), jnp.float32),
    scratch_shapes=[pltpu.VMEM((BC, D_Z), jnp.bfloat16)],
)


def kernel(Z, W_vm, clause_weights, neural_idx, b_idx, clause_idx):
    neg1 = jnp.full((N_PAD,), -1, jnp.int32)
    zero = jnp.zeros((N_PAD,), jnp.float32)
    sel, cvec = _sc_build(neural_idx, b_idx, clause_idx, clause_weights,
                          neg1, zero)
    return _tc_apply(Z, W_vm, sel.reshape(1, N_PAD), cvec.reshape(1, N_PAD))
